# baseline (speedup 1.0000x reference)
import math

import jax
import jax.numpy as jnp
from jax.experimental import pallas as pl
from jax.experimental.pallas import tpu as pltpu

KNN = 16
_INTERPRET = False


def _fps_body(px_ref, py_ref, pz_ref, prow_ref, out_ref, mind_ref,
              *, n_valid, n_out):
    R = px_ref.shape[0]
    row = jax.lax.broadcasted_iota(jnp.int32, (R, 128), 0)
    col = jax.lax.broadcasted_iota(jnp.int32, (R, 128), 1)
    flat = row * 128 + col
    valid = flat < n_valid
    mind_ref[...] = jnp.where(valid, jnp.inf, -jnp.inf)
    out_ref[0:1, 0:1] = jnp.zeros((1, 1), jnp.int32)

    px = px_ref[...]
    py = py_ref[...]
    pz = pz_ref[...]

    def body(i, nxt):
        last = prow_ref[pl.ds(nxt, 1), :]
        lx = last[0:1, 0:1]
        ly = last[0:1, 1:2]
        lz = last[0:1, 2:3]
        dx = px - lx
        dy = py - ly
        dz = pz - lz
        d = (dx * dx + dy * dy) + dz * dz
        mind = jnp.minimum(mind_ref[...], d)
        mind_ref[...] = mind
        m = jnp.max(mind)
        cand = jnp.where(mind == m, flat, jnp.int32(2**30))
        nxt2 = jnp.min(cand)
        out_ref[pl.ds(i, 1), :] = jnp.full((1, 1), nxt2, jnp.int32)
        return nxt2

    jax.lax.fori_loop(1, n_out, body, jnp.int32(0))


def _fps(pos, n_out):
    N = pos.shape[0]
    P = ((N + 127) // 128) * 128
    R = P // 128
    posp = jnp.pad(pos, ((0, P - N), (0, 0)))
    px = posp[:, 0].reshape(R, 128)
    py = posp[:, 1].reshape(R, 128)
    pz = posp[:, 2].reshape(R, 128)
    import functools
    body = functools.partial(_fps_body, n_valid=N, n_out=n_out)
    idx = pl.pallas_call(
        body,
        out_shape=jax.ShapeDtypeStruct((n_out, 1), jnp.int32),
        scratch_shapes=[pltpu.VMEM((R, 128), jnp.float32)],
        interpret=_INTERPRET,
    )(px, py, pz, posp)
    return idx.reshape(n_out)


def _linear(x, p):
    return x @ p["w"] + p["b"]


def _layer_norm(x, p):
    m = jnp.mean(x, axis=-1, keepdims=True)
    v = jnp.var(x, axis=-1, keepdims=True)
    return (x - m) / jnp.sqrt(v + 1e-5) * p["g"] + p["b"]


def _knn_body(prow_ref, pt_ref, out_ref, *, n_valid, k, block_b):
    B = block_b
    P = pt_ref.shape[1]
    i = pl.program_id(0)
    xc = prow_ref[:, 0:1]
    yc = prow_ref[:, 1:2]
    zc = prow_ref[:, 2:3]
    xr = pt_ref[0:1, :]
    yr = pt_ref[1:2, :]
    zr = pt_ref[2:3, :]
    dx = xc - xr
    dy = yc - yr
    dz = zc - zr
    d = (dx * dx + dy * dy) + dz * dz
    col = jax.lax.broadcasted_iota(jnp.int32, (B, P), 1)
    rowg = jax.lax.broadcasted_iota(jnp.int32, (B, P), 0) + i * B
    d = d + jnp.where(col == rowg, jnp.float32(1e10), jnp.float32(0.0))
    d = jnp.where(col < n_valid, d, jnp.inf)
    for r in range(k):
        m = jnp.min(d, axis=1, keepdims=True)
        idx = jnp.min(jnp.where(d == m, col, jnp.int32(2**30)),
                      axis=1, keepdims=True)
        out_ref[:, r:r + 1] = idx
        d = jnp.where(col == idx, jnp.inf, d)


def _knn(pos, k):
    N = pos.shape[0]
    B = 128
    P = ((N + B - 1) // B) * B
    posp = jnp.pad(pos, ((0, P - N), (0, 0)))
    post = posp.T
    import functools
    body = functools.partial(_knn_body, n_valid=N, k=k, block_b=B)
    nbr = pl.pallas_call(
        body,
        grid=(P // B,),
        in_specs=[pl.BlockSpec((B, 3), lambda i: (i, 0)),
                  pl.BlockSpec((3, P), lambda i: (0, 0))],
        out_specs=pl.BlockSpec((B, k), lambda i: (i, 0)),
        out_shape=jax.ShapeDtypeStruct((P, k), jnp.int32),
        interpret=_INTERPRET,
    )(posp, post)
    return nbr[:N]


def _segsoftmax_dense(scores):
    mx = jnp.max(scores, axis=1, keepdims=True)
    mx = jnp.where(jnp.isfinite(mx), mx, 0.0)
    e = jnp.exp(scores - mx)
    s = jnp.sum(e, axis=1, keepdims=True)
    return e / (s + 1e-16)


def _ln_rows(x, g, b):
    m = jnp.mean(x, axis=-1, keepdims=True)
    xc = x - m
    v = jnp.mean(xc * xc, axis=-1, keepdims=True)
    return xc / jnp.sqrt(v + 1e-5) * g + b


def _agt_node_body(x_ref, wf_ref, bf_ref, fg_ref, fb_ref, w1_ref,
                   wq_ref, bq_ref, wr_ref, br_ref,
                   fw1_ref, hi_ref, lo_ref, qn_ref, res_ref, *, has_res):
    xx = x_ref[...]
    f = jnp.maximum(jnp.dot(xx, wf_ref[...],
                            preferred_element_type=jnp.float32)
                    + bf_ref[...], 0.0)
    f = _ln_rows(f, fg_ref[...], fb_ref[...])
    fw1 = jnp.dot(f, w1_ref[...], preferred_element_type=jnp.float32)
    fw1_ref[...] = fw1
    hi = fw1.astype(jnp.bfloat16)
    hi_ref[...] = hi
    lo_ref[...] = (fw1 - hi.astype(jnp.float32)).astype(jnp.bfloat16)
    qn_ref[...] = jnp.dot(f, wq_ref[...],
                          preferred_element_type=jnp.float32) + bq_ref[...]
    if has_res:
        res_ref[...] = jnp.dot(xx, wr_ref[...],
                               preferred_element_type=jnp.float32) + br_ref[...]
    else:
        res_ref[...] = xx


def _agt_edge_body(nbr_ref, hi_ref, lo_ref, fw1_ref, qn_ref, res_ref,
                   ph_ref, plo_ref, pd_ref,
                   w2_ref, bwf_ref, wfg_ref, wfb_ref,
                   wk_ref, bk_ref, wp_ref, bp_ref, pg_ref, pb_ref,
                   lg_ref, lb_ref, out_ref, w_scr, s_scr, *, c, k):
    B = nbr_ref.shape[0]
    P = hi_ref.shape[0]
    colP = jax.lax.broadcasted_iota(jnp.int32, (B, P), 1)
    fw1_d = fw1_ref[...]
    qn_d = qn_ref[...]
    pos_d = pd_ref[...]
    inv = 1.0 / math.sqrt(c)
    for j in range(k):
        srcj = nbr_ref[:, j:j + 1]
        oh = (colP == srcj)
        ohb = oh.astype(jnp.bfloat16)
        g = (jnp.dot(ohb, hi_ref[...], preferred_element_type=jnp.float32)
             + jnp.dot(ohb, lo_ref[...], preferred_element_type=jnp.float32))
        ps = (jnp.dot(ohb, ph_ref[...], preferred_element_type=jnp.float32)
              + jnp.dot(ohb, plo_ref[...], preferred_element_type=jnp.float32))
        dp = pos_d - ps
        pre = (fw1_d - g
               + jnp.dot(dp, w2_ref[...], preferred_element_type=jnp.float32)
               + bwf_ref[...])
        w_e = _ln_rows(jnp.maximum(pre, 0.0), wfg_ref[...], wfb_ref[...])
        pq = _ln_rows(
            jnp.maximum(jnp.dot(dp, wp_ref[...],
                                preferred_element_type=jnp.float32)
                        + bp_ref[...], 0.0),
            pg_ref[...], pb_ref[...])
        q_e = qn_d + pq
        kk = jnp.dot(w_e, wk_ref[...],
                     preferred_element_type=jnp.float32) + bk_ref[...]
        s_scr[:, j:j + 1] = jnp.sum(q_e * kk, axis=1, keepdims=True) * inv
        w_scr[j * B:(j + 1) * B, :] = w_e
    s = s_scr[...]
    mx = jnp.max(s, axis=1, keepdims=True)
    e = jnp.exp(s - mx)
    attn = e / (jnp.sum(e, axis=1, keepdims=True) + 1e-16)
    agg = jnp.zeros((B, c), jnp.float32)
    for j in range(k):
        agg = agg + attn[:, j:j + 1] * w_scr[j * B:(j + 1) * B, :]
    out_ref[...] = _ln_rows(agg + res_ref[...], lg_ref[...], lb_ref[...])


def _agt_block(p, x, pos, nbr):
    N, k = nbr.shape
    cin = x.shape[1]
    c = p["feat"]["w"].shape[1]
    B = 128
    P = ((N + B - 1) // B) * B
    import functools
    has_res = "res" in p
    xp = jnp.pad(x, ((0, P - N), (0, 0)))
    w1 = p["wf"]["w"][:c]
    w2 = p["wf"]["w"][c:]
    wr = p["res"]["w"] if has_res else jnp.zeros((cin, c), jnp.float32)
    br = p["res"]["b"] if has_res else jnp.zeros((c,), jnp.float32)
    row = lambda v: v.reshape(1, -1)
    nblk = P // B
    node = pl.pallas_call(
        functools.partial(_agt_node_body, has_res=has_res),
        grid=(nblk,),
        in_specs=[pl.BlockSpec((B, cin), lambda i: (i, 0))]
        + [pl.BlockSpec(w.shape, lambda i: tuple(0 for _ in w.shape))
           for w in (p["feat"]["w"], row(p["feat"]["b"]),
                     row(p["feat_ln"]["g"]), row(p["feat_ln"]["b"]), w1,
                     p["q"]["w"], row(p["q"]["b"]), wr, row(br))],
        out_specs=[pl.BlockSpec((B, c), lambda i: (i, 0))] * 5,
        out_shape=[jax.ShapeDtypeStruct((P, c), jnp.float32),
                   jax.ShapeDtypeStruct((P, c), jnp.bfloat16),
                   jax.ShapeDtypeStruct((P, c), jnp.bfloat16),
                   jax.ShapeDtypeStruct((P, c), jnp.float32),
                   jax.ShapeDtypeStruct((P, c), jnp.float32)],
        interpret=_INTERPRET,
    )(xp, p["feat"]["w"], row(p["feat"]["b"]),
      row(p["feat_ln"]["g"]), row(p["feat_ln"]["b"]), w1,
      p["q"]["w"], row(p["q"]["b"]), wr, row(br))
    fw1, hi, lo, qn, res = node

    posp = jnp.pad(pos, ((0, P - N), (0, 0)))
    ph = posp.astype(jnp.bfloat16)
    plo = (posp - ph.astype(jnp.float32)).astype(jnp.bfloat16)
    nbrp = jnp.pad(nbr, ((0, P - N), (0, 0)))
    out = pl.pallas_call(
        functools.partial(_agt_edge_body, c=c, k=k),
        grid=(nblk,),
        in_specs=[pl.BlockSpec((B, k), lambda i: (i, 0)),
                  pl.BlockSpec((P, c), lambda i: (0, 0)),
                  pl.BlockSpec((P, c), lambda i: (0, 0)),
                  pl.BlockSpec((B, c), lambda i: (i, 0)),
                  pl.BlockSpec((B, c), lambda i: (i, 0)),
                  pl.BlockSpec((B, c), lambda i: (i, 0)),
                  pl.BlockSpec((P, 3), lambda i: (0, 0)),
                  pl.BlockSpec((P, 3), lambda i: (0, 0)),
                  pl.BlockSpec((B, 3), lambda i: (i, 0))]
        + [pl.BlockSpec(w.shape, lambda i: tuple(0 for _ in w.shape))
           for w in (w2, row(p["wf"]["b"]),
                     row(p["wf_ln"]["g"]), row(p["wf_ln"]["b"]),
                     p["k"]["w"], row(p["k"]["b"]),
                     p["pos"]["w"], row(p["pos"]["b"]),
                     row(p["pos_ln"]["g"]), row(p["pos_ln"]["b"]),
                     row(p["final_ln"]["g"]), row(p["final_ln"]["b"]))],
        out_specs=pl.BlockSpec((B, c), lambda i: (i, 0)),
        out_shape=jax.ShapeDtypeStruct((P, c), jnp.float32),
        scratch_shapes=[pltpu.VMEM((k * B, c), jnp.float32),
                        pltpu.VMEM((B, k), jnp.float32)],
        interpret=_INTERPRET,
    )(nbrp, hi, lo, fw1, qn, res, ph, plo, posp,
      w2, row(p["wf"]["b"]),
      row(p["wf_ln"]["g"]), row(p["wf_ln"]["b"]),
      p["k"]["w"], row(p["k"]["b"]),
      p["pos"]["w"], row(p["pos"]["b"]),
      row(p["pos_ln"]["g"]), row(p["pos_ln"]["b"]),
      row(p["final_ln"]["g"]), row(p["final_ln"]["b"]))
    return out[:N]


def _virtual_node(p, x):
    gc = jnp.mean(x, axis=0, keepdims=True)
    gc = _layer_norm(_linear(gc, p["agg"]), p["ln"])
    return x + _linear(gc, p["dist"])


def kernel(x, pos, labels, params):
    features = [x]
    positions = [pos]
    slabels = [labels]
    h = _layer_norm(jax.nn.relu(_linear(x, params["stage0"]["lin"])),
                    params["stage0"]["ln"])
    h = _virtual_node(params["vn0"], h)
    features.append(h); positions.append(pos); slabels.append(labels)
    cur_pos, cur_lab = pos, labels
    for stage_key, vn_key, ratio in (("stage1", "vn1", 0.25),
                                     ("stage2", "vn2", 0.25)):
        n = int(h.shape[0] * ratio)
        idx = _fps(cur_pos, n)
        h = h[idx]; cur_pos = cur_pos[idx]; cur_lab = cur_lab[idx]
        k_safe = min(KNN, h.shape[0] - 1)
        nbr = _knn(cur_pos, k_safe)
        for blk in params[stage_key]:
            h = _agt_block(blk, h, cur_pos, nbr)
        h = _virtual_node(params[vn_key], h)
        features.append(h); positions.append(cur_pos); slabels.append(cur_lab)
    return (tuple(features), tuple(positions), tuple(slabels))


# speedup vs baseline: 1.1443x; 1.1443x over previous
import math

import jax
import jax.numpy as jnp
from jax.experimental import pallas as pl
from jax.experimental.pallas import tpu as pltpu

KNN = 16
_INTERPRET = False


def _fps_body(px_ref, py_ref, pz_ref, prow_ref, out_ref, mind_ref,
              *, n_valid, n_out):
    R = px_ref.shape[0]
    row = jax.lax.broadcasted_iota(jnp.int32, (R, 128), 0)
    col = jax.lax.broadcasted_iota(jnp.int32, (R, 128), 1)
    flat = row * 128 + col
    valid = flat < n_valid
    mind_ref[...] = jnp.where(valid, jnp.inf, -jnp.inf)
    out_ref[0:1, 0:1] = jnp.zeros((1, 1), jnp.int32)

    px = px_ref[...]
    py = py_ref[...]
    pz = pz_ref[...]

    def body(i, nxt):
        last = prow_ref[pl.ds(nxt, 1), :]
        lx = last[0:1, 0:1]
        ly = last[0:1, 1:2]
        lz = last[0:1, 2:3]
        dx = px - lx
        dy = py - ly
        dz = pz - lz
        d = (dx * dx + dy * dy) + dz * dz
        mind = jnp.minimum(mind_ref[...], d)
        mind_ref[...] = mind
        m = jnp.max(mind)
        cand = jnp.where(mind == m, flat, jnp.int32(2**30))
        nxt2 = jnp.min(cand)
        out_ref[pl.ds(i, 1), :] = jnp.full((1, 1), nxt2, jnp.int32)
        return nxt2

    jax.lax.fori_loop(1, n_out, body, jnp.int32(0))


def _fps(pos, n_out):
    N = pos.shape[0]
    P = ((N + 127) // 128) * 128
    R = P // 128
    posp = jnp.pad(pos, ((0, P - N), (0, 0)))
    px = posp[:, 0].reshape(R, 128)
    py = posp[:, 1].reshape(R, 128)
    pz = posp[:, 2].reshape(R, 128)
    import functools
    body = functools.partial(_fps_body, n_valid=N, n_out=n_out)
    idx = pl.pallas_call(
        body,
        out_shape=jax.ShapeDtypeStruct((n_out, 1), jnp.int32),
        scratch_shapes=[pltpu.VMEM((R, 128), jnp.float32)],
        interpret=_INTERPRET,
    )(px, py, pz, posp)
    return idx.reshape(n_out)


def _linear(x, p):
    return x @ p["w"] + p["b"]


def _layer_norm(x, p):
    m = jnp.mean(x, axis=-1, keepdims=True)
    v = jnp.var(x, axis=-1, keepdims=True)
    return (x - m) / jnp.sqrt(v + 1e-5) * p["g"] + p["b"]


def _knn_body(prow_ref, pt_ref, out_ref, *, n_valid, k, block_b):
    B = block_b
    P = pt_ref.shape[1]
    i = pl.program_id(0)
    xc = prow_ref[:, 0:1]
    yc = prow_ref[:, 1:2]
    zc = prow_ref[:, 2:3]
    xr = pt_ref[0:1, :]
    yr = pt_ref[1:2, :]
    zr = pt_ref[2:3, :]
    dx = xc - xr
    dy = yc - yr
    dz = zc - zr
    d = (dx * dx + dy * dy) + dz * dz
    col = jax.lax.broadcasted_iota(jnp.int32, (B, P), 1)
    rowg = jax.lax.broadcasted_iota(jnp.int32, (B, P), 0) + i * B
    d = d + jnp.where(col == rowg, jnp.float32(1e10), jnp.float32(0.0))
    d = jnp.where(col < n_valid, d, jnp.inf)
    for r in range(k):
        m = jnp.min(d, axis=1, keepdims=True)
        idx = jnp.min(jnp.where(d == m, col, jnp.int32(2**30)),
                      axis=1, keepdims=True)
        out_ref[:, r:r + 1] = idx
        d = jnp.where(col == idx, jnp.inf, d)


def _knn(pos, k):
    N = pos.shape[0]
    B = 128
    P = ((N + B - 1) // B) * B
    posp = jnp.pad(pos, ((0, P - N), (0, 0)))
    post = posp.T
    import functools
    body = functools.partial(_knn_body, n_valid=N, k=k, block_b=B)
    nbr = pl.pallas_call(
        body,
        grid=(P // B,),
        in_specs=[pl.BlockSpec((B, 3), lambda i: (i, 0)),
                  pl.BlockSpec((3, P), lambda i: (0, 0))],
        out_specs=pl.BlockSpec((B, k), lambda i: (i, 0)),
        out_shape=jax.ShapeDtypeStruct((P, k), jnp.int32),
        interpret=_INTERPRET,
    )(posp, post)
    return nbr[:N]


def _segsoftmax_dense(scores):
    mx = jnp.max(scores, axis=1, keepdims=True)
    mx = jnp.where(jnp.isfinite(mx), mx, 0.0)
    e = jnp.exp(scores - mx)
    s = jnp.sum(e, axis=1, keepdims=True)
    return e / (s + 1e-16)


def _ln_rows(x, g, b):
    m = jnp.mean(x, axis=-1, keepdims=True)
    xc = x - m
    v = jnp.mean(xc * xc, axis=-1, keepdims=True)
    return xc / jnp.sqrt(v + 1e-5) * g + b


def _agt_node_body(x_ref, wf_ref, bf_ref, fg_ref, fb_ref, w1_ref,
                   wq_ref, bq_ref, wr_ref, br_ref,
                   fw1_ref, hi_ref, lo_ref, qn_ref, res_ref, *, has_res):
    xx = x_ref[...]
    f = jnp.maximum(jnp.dot(xx, wf_ref[...],
                            preferred_element_type=jnp.float32)
                    + bf_ref[...], 0.0)
    f = _ln_rows(f, fg_ref[...], fb_ref[...])
    fw1 = jnp.dot(f, w1_ref[...], preferred_element_type=jnp.float32)
    fw1_ref[...] = fw1
    hi = fw1.astype(jnp.bfloat16)
    hi_ref[...] = hi
    lo_ref[...] = (fw1 - hi.astype(jnp.float32)).astype(jnp.bfloat16)
    qn_ref[...] = jnp.dot(f, wq_ref[...],
                          preferred_element_type=jnp.float32) + bq_ref[...]
    if has_res:
        res_ref[...] = jnp.dot(xx, wr_ref[...],
                               preferred_element_type=jnp.float32) + br_ref[...]
    else:
        res_ref[...] = xx


def _agt_edge_body(nbr_ref, hi_ref, lo_ref, fw1_ref, qn_ref, res_ref,
                   ph_ref, plo_ref, pd_ref,
                   w2_ref, bwf_ref, wfg_ref, wfb_ref,
                   wk_ref, bk_ref, wp_ref, bp_ref, pg_ref, pb_ref,
                   lg_ref, lb_ref, out_ref, oh_scr, dup_scr, *, c, k):
    B = nbr_ref.shape[0]
    P = hi_ref.shape[0]
    colP = jax.lax.broadcasted_iota(jnp.int32, (B, P), 1)
    fw1_d = fw1_ref[...]
    qn_d = qn_ref[...]
    pos_d = pd_ref[...]
    inv = 1.0 / math.sqrt(c)
    for j in range(k):
        srcj = nbr_ref[:, j:j + 1]
        oh_scr[j * B:(j + 1) * B, :] = (colP == srcj).astype(jnp.bfloat16)
        dup_scr[j * B:(j + 1) * B, 0:c] = fw1_d
        dup_scr[j * B:(j + 1) * B, c:2 * c] = qn_d
        dup_scr[j * B:(j + 1) * B, 2 * c:2 * c + 3] = pos_d
    oh = oh_scr[...]
    g = (jnp.dot(oh, hi_ref[...], preferred_element_type=jnp.float32)
         + jnp.dot(oh, lo_ref[...], preferred_element_type=jnp.float32))
    ps = (jnp.dot(oh, ph_ref[...], preferred_element_type=jnp.float32)
          + jnp.dot(oh, plo_ref[...], preferred_element_type=jnp.float32))
    dp = dup_scr[:, 2 * c:2 * c + 3] - ps
    pre = (dup_scr[:, 0:c] - g
           + jnp.dot(dp, w2_ref[...], preferred_element_type=jnp.float32)
           + bwf_ref[...])
    w_e = _ln_rows(jnp.maximum(pre, 0.0), wfg_ref[...], wfb_ref[...])
    pq = _ln_rows(
        jnp.maximum(jnp.dot(dp, wp_ref[...],
                            preferred_element_type=jnp.float32)
                    + bp_ref[...], 0.0),
        pg_ref[...], pb_ref[...])
    q_e = dup_scr[:, c:2 * c] + pq
    kk = jnp.dot(w_e, wk_ref[...],
                 preferred_element_type=jnp.float32) + bk_ref[...]
    score = jnp.sum(q_e * kk, axis=1, keepdims=True) * inv
    mx = score[0:B, :]
    for j in range(1, k):
        mx = jnp.maximum(mx, score[j * B:(j + 1) * B, :])
    ssum = jnp.zeros((B, 1), jnp.float32)
    agg = jnp.zeros((B, c), jnp.float32)
    for j in range(k):
        e = jnp.exp(score[j * B:(j + 1) * B, :] - mx)
        ssum = ssum + e
        agg = agg + e * w_e[j * B:(j + 1) * B, :]
    agg = agg / (ssum + 1e-16)
    out_ref[...] = _ln_rows(agg + res_ref[...], lg_ref[...], lb_ref[...])


def _agt_block(p, x, pos, nbr):
    N, k = nbr.shape
    cin = x.shape[1]
    c = p["feat"]["w"].shape[1]
    B = 128
    P = ((N + B - 1) // B) * B
    import functools
    has_res = "res" in p
    xp = jnp.pad(x, ((0, P - N), (0, 0)))
    w1 = p["wf"]["w"][:c]
    w2 = p["wf"]["w"][c:]
    wr = p["res"]["w"] if has_res else jnp.zeros((cin, c), jnp.float32)
    br = p["res"]["b"] if has_res else jnp.zeros((c,), jnp.float32)
    row = lambda v: v.reshape(1, -1)
    nblk = P // B
    node = pl.pallas_call(
        functools.partial(_agt_node_body, has_res=has_res),
        grid=(nblk,),
        in_specs=[pl.BlockSpec((B, cin), lambda i: (i, 0))]
        + [pl.BlockSpec(w.shape, lambda i: tuple(0 for _ in w.shape))
           for w in (p["feat"]["w"], row(p["feat"]["b"]),
                     row(p["feat_ln"]["g"]), row(p["feat_ln"]["b"]), w1,
                     p["q"]["w"], row(p["q"]["b"]), wr, row(br))],
        out_specs=[pl.BlockSpec((B, c), lambda i: (i, 0))] * 5,
        out_shape=[jax.ShapeDtypeStruct((P, c), jnp.float32),
                   jax.ShapeDtypeStruct((P, c), jnp.bfloat16),
                   jax.ShapeDtypeStruct((P, c), jnp.bfloat16),
                   jax.ShapeDtypeStruct((P, c), jnp.float32),
                   jax.ShapeDtypeStruct((P, c), jnp.float32)],
        interpret=_INTERPRET,
    )(xp, p["feat"]["w"], row(p["feat"]["b"]),
      row(p["feat_ln"]["g"]), row(p["feat_ln"]["b"]), w1,
      p["q"]["w"], row(p["q"]["b"]), wr, row(br))
    fw1, hi, lo, qn, res = node

    posp = jnp.pad(pos, ((0, P - N), (0, 0)))
    ph = posp.astype(jnp.bfloat16)
    plo = (posp - ph.astype(jnp.float32)).astype(jnp.bfloat16)
    nbrp = jnp.pad(nbr, ((0, P - N), (0, 0)))
    out = pl.pallas_call(
        functools.partial(_agt_edge_body, c=c, k=k),
        grid=(nblk,),
        in_specs=[pl.BlockSpec((B, k), lambda i: (i, 0)),
                  pl.BlockSpec((P, c), lambda i: (0, 0)),
                  pl.BlockSpec((P, c), lambda i: (0, 0)),
                  pl.BlockSpec((B, c), lambda i: (i, 0)),
                  pl.BlockSpec((B, c), lambda i: (i, 0)),
                  pl.BlockSpec((B, c), lambda i: (i, 0)),
                  pl.BlockSpec((P, 3), lambda i: (0, 0)),
                  pl.BlockSpec((P, 3), lambda i: (0, 0)),
                  pl.BlockSpec((B, 3), lambda i: (i, 0))]
        + [pl.BlockSpec(w.shape, lambda i: tuple(0 for _ in w.shape))
           for w in (w2, row(p["wf"]["b"]),
                     row(p["wf_ln"]["g"]), row(p["wf_ln"]["b"]),
                     p["k"]["w"], row(p["k"]["b"]),
                     p["pos"]["w"], row(p["pos"]["b"]),
                     row(p["pos_ln"]["g"]), row(p["pos_ln"]["b"]),
                     row(p["final_ln"]["g"]), row(p["final_ln"]["b"]))],
        out_specs=pl.BlockSpec((B, c), lambda i: (i, 0)),
        out_shape=jax.ShapeDtypeStruct((P, c), jnp.float32),
        scratch_shapes=[pltpu.VMEM((k * B, P), jnp.bfloat16),
                        pltpu.VMEM((k * B, 2 * c + 3), jnp.float32)],
        interpret=_INTERPRET,
    )(nbrp, hi, lo, fw1, qn, res, ph, plo, posp,
      w2, row(p["wf"]["b"]),
      row(p["wf_ln"]["g"]), row(p["wf_ln"]["b"]),
      p["k"]["w"], row(p["k"]["b"]),
      p["pos"]["w"], row(p["pos"]["b"]),
      row(p["pos_ln"]["g"]), row(p["pos_ln"]["b"]),
      row(p["final_ln"]["g"]), row(p["final_ln"]["b"]))
    return out[:N]


def _virtual_node(p, x):
    gc = jnp.mean(x, axis=0, keepdims=True)
    gc = _layer_norm(_linear(gc, p["agg"]), p["ln"])
    return x + _linear(gc, p["dist"])


def kernel(x, pos, labels, params):
    features = [x]
    positions = [pos]
    slabels = [labels]
    h = _layer_norm(jax.nn.relu(_linear(x, params["stage0"]["lin"])),
                    params["stage0"]["ln"])
    h = _virtual_node(params["vn0"], h)
    features.append(h); positions.append(pos); slabels.append(labels)
    cur_pos, cur_lab = pos, labels
    for stage_key, vn_key, ratio in (("stage1", "vn1", 0.25),
                                     ("stage2", "vn2", 0.25)):
        n = int(h.shape[0] * ratio)
        idx = _fps(cur_pos, n)
        h = h[idx]; cur_pos = cur_pos[idx]; cur_lab = cur_lab[idx]
        k_safe = min(KNN, h.shape[0] - 1)
        nbr = _knn(cur_pos, k_safe)
        for blk in params[stage_key]:
            h = _agt_block(blk, h, cur_pos, nbr)
        h = _virtual_node(params[vn_key], h)
        features.append(h); positions.append(cur_pos); slabels.append(cur_lab)
    return (tuple(features), tuple(positions), tuple(slabels))
